# baseline (speedup 1.0000x reference)
import functools

import jax
import jax.numpy as jnp
from jax import lax
from jax.experimental import pallas as pl
from jax.experimental.pallas import tpu as pltpu
from jax.experimental.pallas import tpu_sc as plsc

N = 10000
NP = 10112
E = 160000
ER = 1280
ERP = 1344
RPW_DEG = ER // 32
RPS = ER // 16
R0 = 58
R0A = 56
RMAX = 64
NPS = NP // 16


@functools.cache
def _make_deg():
    mesh = plsc.VectorSubcoreMesh(core_axis_name="c", subcore_axis_name="s")

    @functools.partial(
        pl.kernel, mesh=mesh,
        out_type=jax.ShapeDtypeStruct((2, NP, 128), jnp.float32),
        scratch_types=[
            pltpu.VMEM((RPW_DEG, 128), jnp.int32),
            pltpu.VMEM((128, 128), jnp.float32),
            pltpu.VMEM_SHARED((NP, 128), jnp.float32),
            pltpu.SemaphoreType.DMA,
        ],
    )
    def k(dst_hbm, ones_hbm, zeros_hbm, out_hbm, dst_v, ones_v, accum, sem):
        c = lax.axis_index("c")
        s = lax.axis_index("s")
        w = c * 16 + s
        base = s * NPS
        pltpu.sync_copy(zeros_hbm.at[pl.ds(base, NPS)], accum.at[pl.ds(base, NPS)])
        pltpu.sync_copy(dst_hbm.at[pl.ds(w * RPW_DEG, RPW_DEG)], dst_v)
        pltpu.sync_copy(ones_hbm, ones_v)
        plsc.subcore_barrier()

        def body(j, carry):
            pltpu.async_copy(ones_v, accum.at[dst_v.at[j]], sem, add=True)
            return carry

        lax.fori_loop(0, RPW_DEG, body, 0)

        def drain(j, carry):
            pltpu.make_async_copy(ones_v, accum.at[dst_v.at[0]], sem).wait()
            return carry

        lax.fori_loop(0, RPW_DEG, drain, 0)
        plsc.subcore_barrier()
        pltpu.sync_copy(accum.at[pl.ds(base, NPS)], out_hbm.at[c, pl.ds(base, NPS)])

    return k


@functools.cache
def _make_agg():
    mesh = plsc.VectorSubcoreMesh(core_axis_name="c", subcore_axis_name="s")

    @functools.partial(
        pl.kernel, mesh=mesh,
        out_type=jax.ShapeDtypeStruct((2, NP, 128), jnp.float32),
        scratch_types=[
            pltpu.VMEM((RMAX, 128), jnp.int32),
            pltpu.VMEM((RMAX, 128), jnp.int32),
            pltpu.VMEM((128, 128), jnp.float32),
            pltpu.VMEM((128, 128), jnp.float32),
            pltpu.VMEM_SHARED((NP, 128), jnp.float32),
            pltpu.SemaphoreType.DMA,
            pltpu.SemaphoreType.DMA,
        ],
    )
    def k(g_hbm, src_hbm, dst_hbm, zeros_hbm, out_hbm,
          src_v, dst_v, buf_a, buf_b, accum, sem_a, sem_b):
        c = lax.axis_index("c")
        s = lax.axis_index("s")
        base = s * NPS
        row0 = s * RPS + c * R0A
        off = c * (R0 - R0A)
        nr = R0 + c * (RPS - 2 * R0)
        pltpu.sync_copy(zeros_hbm.at[pl.ds(base, NPS)], accum.at[pl.ds(base, NPS)])
        pltpu.sync_copy(src_hbm.at[pl.ds(row0, RMAX)], src_v)
        pltpu.sync_copy(dst_hbm.at[pl.ds(row0, RMAX)], dst_v)
        plsc.subcore_barrier()

        def gather(j, buf, sem):
            pltpu.async_copy(g_hbm.at[src_v.at[off + j]], buf, sem)

        def gwait(buf, sem):
            pltpu.make_async_copy(g_hbm.at[src_v.at[0]], buf, sem).wait()

        def scatter(j, buf):
            pltpu.sync_copy(buf, accum.at[dst_v.at[off + j]], add=True)

        def body(p, carry):
            j0 = 2 * p
            gather(j0 + 1, buf_b, sem_b)
            gwait(buf_a, sem_a)
            scatter(j0, buf_a)
            jn = lax.min(j0 + 2, nr - 2)
            gather(jn, buf_a, sem_a)
            gwait(buf_b, sem_b)
            scatter(j0 + 1, buf_b)
            return carry

        gather(0, buf_a, sem_a)
        lax.fori_loop(0, nr // 2, body, 0)
        gwait(buf_a, sem_a)

        plsc.subcore_barrier()
        pltpu.sync_copy(accum.at[pl.ds(base, NPS)], out_hbm.at[c, pl.ds(base, NPS)])

    return k


_R = 5000


def _dinv_block(deg_ref):
    p = deg_ref[...]
    return lax.rsqrt(p[0, :, 0:1] + p[1, :, 0:1] + 1.0)


def _mm1(degp, x, W1):
    def body(deg_ref, x_ref, w_ref, out_ref):
        h = jnp.dot(x_ref[...], w_ref[...], preferred_element_type=jnp.float32)
        out_ref[...] = h * _dinv_block(deg_ref)

    return pl.pallas_call(
        body,
        grid=(2,),
        in_specs=[
            pl.BlockSpec((2, _R, 128), lambda i: (0, i, 0)),
            pl.BlockSpec((_R, 256), lambda i: (i, 0)),
            pl.BlockSpec((256, 128), lambda i: (0, 0)),
        ],
        out_specs=pl.BlockSpec((_R, 128), lambda i: (i, 0)),
        out_shape=jax.ShapeDtypeStruct((NP, 128), jnp.float32),
    )(degp, x, W1)


def _mid(degp, agg1, g1, b1):
    def body(deg_ref, a_ref, g_ref, b_ref, out_ref):
        dinv = _dinv_block(deg_ref)
        a = a_ref[...]
        x1 = a[0] + a[1] + g_ref[...]
        out_ref[...] = dinv * jnp.maximum(dinv * x1 + b_ref[...], 0.0)

    return pl.pallas_call(
        body,
        grid=(2,),
        in_specs=[
            pl.BlockSpec((2, _R, 128), lambda i: (0, i, 0)),
            pl.BlockSpec((2, _R, 128), lambda i: (0, i, 0)),
            pl.BlockSpec((_R, 128), lambda i: (i, 0)),
            pl.BlockSpec((1, 128), lambda i: (0, 0)),
        ],
        out_specs=pl.BlockSpec((_R, 128), lambda i: (i, 0)),
        out_shape=jax.ShapeDtypeStruct((NP, 128), jnp.float32),
    )(degp, agg1, g1, b1.reshape(1, 128))


def _final(degp, agg2, y, W2, b2):
    def body(deg_ref, a_ref, y_ref, w_ref, b_ref, out_ref):
        dinv = _dinv_block(deg_ref)
        a = a_ref[...]
        z = dinv * (a[0] + a[1] + y_ref[...])
        out_ref[...] = (
            jnp.dot(z, w_ref[...], preferred_element_type=jnp.float32)
            + b_ref[...]
        )

    return pl.pallas_call(
        body,
        grid=(2,),
        in_specs=[
            pl.BlockSpec((2, _R, 128), lambda i: (0, i, 0)),
            pl.BlockSpec((2, _R, 128), lambda i: (0, i, 0)),
            pl.BlockSpec((_R, 128), lambda i: (i, 0)),
            pl.BlockSpec((128, 256), lambda i: (0, 0)),
            pl.BlockSpec((1, 256), lambda i: (0, 0)),
        ],
        out_specs=pl.BlockSpec((_R, 256), lambda i: (i, 0)),
        out_shape=jax.ShapeDtypeStruct((N, 256), jnp.float32),
    )(degp, agg2, y, W2, b2.reshape(1, 256))


def kernel(x, edge_index, W1, b1, W2, b2):
    src = edge_index[0].astype(jnp.int32)
    dst = edge_index[1].astype(jnp.int32)
    pad = jnp.full((ERP * 128 - E,), N, jnp.int32)
    srcp = jnp.concatenate([src, pad]).reshape(ERP, 128)
    dstp = jnp.concatenate([dst, pad]).reshape(ERP, 128)

    ones = jnp.ones((128, 128), jnp.float32)
    z128 = jnp.zeros((NP, 128), jnp.float32)

    degp = _make_deg()(dstp, ones, z128)
    g1 = _mm1(degp, x, W1)
    agg1 = _make_agg()(g1, srcp, dstp, z128)
    y = _mid(degp, agg1, g1, b1)
    agg2 = _make_agg()(y, srcp, dstp, z128)
    return _final(degp, agg2, y, W2, b2)


# speedup vs baseline: 1.0262x; 1.0262x over previous
import functools

import jax
import jax.numpy as jnp
from jax import lax
from jax.experimental import pallas as pl
from jax.experimental.pallas import tpu as pltpu
from jax.experimental.pallas import tpu_sc as plsc

N = 10000
NP = 10112
E = 160000
ER = 1280
ERP = 1344
RPW_DEG = ER // 32
RPS = ER // 16
R0 = 62
R0A = 56
RMAX = 64
NPS = NP // 16


@functools.cache
def _make_deg():
    mesh = plsc.VectorSubcoreMesh(core_axis_name="c", subcore_axis_name="s")

    @functools.partial(
        pl.kernel, mesh=mesh,
        out_type=jax.ShapeDtypeStruct((2, NP, 128), jnp.float32),
        scratch_types=[
            pltpu.VMEM((RPW_DEG, 128), jnp.int32),
            pltpu.VMEM((128, 128), jnp.float32),
            pltpu.VMEM_SHARED((NP, 128), jnp.float32),
            pltpu.SemaphoreType.DMA,
        ],
    )
    def k(dst_hbm, ones_hbm, zeros_hbm, out_hbm, dst_v, ones_v, accum, sem):
        c = lax.axis_index("c")
        s = lax.axis_index("s")
        w = c * 16 + s
        base = s * NPS
        pltpu.sync_copy(zeros_hbm.at[pl.ds(base, NPS)], accum.at[pl.ds(base, NPS)])
        pltpu.sync_copy(dst_hbm.at[pl.ds(w * RPW_DEG, RPW_DEG)], dst_v)
        pltpu.sync_copy(ones_hbm, ones_v)
        plsc.subcore_barrier()

        def body(j, carry):
            pltpu.async_copy(ones_v, accum.at[dst_v.at[j]], sem, add=True)
            return carry

        lax.fori_loop(0, RPW_DEG, body, 0)

        def drain(j, carry):
            pltpu.make_async_copy(ones_v, accum.at[dst_v.at[0]], sem).wait()
            return carry

        lax.fori_loop(0, RPW_DEG, drain, 0)
        plsc.subcore_barrier()
        pltpu.sync_copy(accum.at[pl.ds(base, NPS)], out_hbm.at[c, pl.ds(base, NPS)])

    return k


@functools.cache
def _make_agg():
    mesh = plsc.VectorSubcoreMesh(core_axis_name="c", subcore_axis_name="s")

    @functools.partial(
        pl.kernel, mesh=mesh,
        out_type=jax.ShapeDtypeStruct((2, NP, 128), jnp.float32),
        scratch_types=[
            pltpu.VMEM((RMAX, 128), jnp.int32),
            pltpu.VMEM((RMAX, 128), jnp.int32),
            pltpu.VMEM((128, 128), jnp.float32),
            pltpu.VMEM((128, 128), jnp.float32),
            pltpu.VMEM_SHARED((NP, 128), jnp.float32),
            pltpu.SemaphoreType.DMA,
            pltpu.SemaphoreType.DMA,
        ],
    )
    def k(g_hbm, src_hbm, dst_hbm, zeros_hbm, out_hbm,
          src_v, dst_v, buf_a, buf_b, accum, sem_a, sem_b):
        c = lax.axis_index("c")
        s = lax.axis_index("s")
        base = s * NPS
        row0 = s * RPS + c * R0A
        off = c * (R0 - R0A)
        nr = R0 + c * (RPS - 2 * R0)
        pltpu.sync_copy(zeros_hbm.at[pl.ds(base, NPS)], accum.at[pl.ds(base, NPS)])
        pltpu.sync_copy(src_hbm.at[pl.ds(row0, RMAX)], src_v)
        pltpu.sync_copy(dst_hbm.at[pl.ds(row0, RMAX)], dst_v)
        plsc.subcore_barrier()

        def gather(j, buf, sem):
            pltpu.async_copy(g_hbm.at[src_v.at[off + j]], buf, sem)

        def gwait(buf, sem):
            pltpu.make_async_copy(g_hbm.at[src_v.at[0]], buf, sem).wait()

        def scatter(j, buf):
            pltpu.sync_copy(buf, accum.at[dst_v.at[off + j]], add=True)

        def body(p, carry):
            j0 = 2 * p
            gather(j0 + 1, buf_b, sem_b)
            gwait(buf_a, sem_a)
            scatter(j0, buf_a)
            jn = lax.min(j0 + 2, nr - 2)
            gather(jn, buf_a, sem_a)
            gwait(buf_b, sem_b)
            scatter(j0 + 1, buf_b)
            return carry

        gather(0, buf_a, sem_a)
        lax.fori_loop(0, nr // 2, body, 0)
        gwait(buf_a, sem_a)

        plsc.subcore_barrier()
        pltpu.sync_copy(accum.at[pl.ds(base, NPS)], out_hbm.at[c, pl.ds(base, NPS)])

    return k


_R = 5000


def _dinv_block(deg_ref):
    p = deg_ref[...]
    return lax.rsqrt(p[0, :, 0:1] + p[1, :, 0:1] + 1.0)


def _mm1(degp, x, W1):
    def body(deg_ref, x_ref, w_ref, out_ref):
        h = jnp.dot(x_ref[...], w_ref[...], preferred_element_type=jnp.float32)
        out_ref[...] = h * _dinv_block(deg_ref)

    return pl.pallas_call(
        body,
        grid=(2,),
        in_specs=[
            pl.BlockSpec((2, _R, 128), lambda i: (0, i, 0)),
            pl.BlockSpec((_R, 256), lambda i: (i, 0)),
            pl.BlockSpec((256, 128), lambda i: (0, 0)),
        ],
        out_specs=pl.BlockSpec((_R, 128), lambda i: (i, 0)),
        out_shape=jax.ShapeDtypeStruct((NP, 128), jnp.float32),
    )(degp, x, W1)


def _mid(degp, agg1, g1, b1):
    def body(deg_ref, a_ref, g_ref, b_ref, out_ref):
        dinv = _dinv_block(deg_ref)
        a = a_ref[...]
        x1 = a[0] + a[1] + g_ref[...]
        out_ref[...] = dinv * jnp.maximum(dinv * x1 + b_ref[...], 0.0)

    return pl.pallas_call(
        body,
        grid=(2,),
        in_specs=[
            pl.BlockSpec((2, _R, 128), lambda i: (0, i, 0)),
            pl.BlockSpec((2, _R, 128), lambda i: (0, i, 0)),
            pl.BlockSpec((_R, 128), lambda i: (i, 0)),
            pl.BlockSpec((1, 128), lambda i: (0, 0)),
        ],
        out_specs=pl.BlockSpec((_R, 128), lambda i: (i, 0)),
        out_shape=jax.ShapeDtypeStruct((NP, 128), jnp.float32),
    )(degp, agg1, g1, b1.reshape(1, 128))


def _final(degp, agg2, y, W2, b2):
    def body(deg_ref, a_ref, y_ref, w_ref, b_ref, out_ref):
        dinv = _dinv_block(deg_ref)
        a = a_ref[...]
        z = dinv * (a[0] + a[1] + y_ref[...])
        out_ref[...] = (
            jnp.dot(z, w_ref[...], preferred_element_type=jnp.float32)
            + b_ref[...]
        )

    return pl.pallas_call(
        body,
        grid=(2,),
        in_specs=[
            pl.BlockSpec((2, _R, 128), lambda i: (0, i, 0)),
            pl.BlockSpec((2, _R, 128), lambda i: (0, i, 0)),
            pl.BlockSpec((_R, 128), lambda i: (i, 0)),
            pl.BlockSpec((128, 256), lambda i: (0, 0)),
            pl.BlockSpec((1, 256), lambda i: (0, 0)),
        ],
        out_specs=pl.BlockSpec((_R, 256), lambda i: (i, 0)),
        out_shape=jax.ShapeDtypeStruct((N, 256), jnp.float32),
    )(degp, agg2, y, W2, b2.reshape(1, 256))


def kernel(x, edge_index, W1, b1, W2, b2):
    src = edge_index[0].astype(jnp.int32)
    dst = edge_index[1].astype(jnp.int32)
    pad = jnp.full((ERP * 128 - E,), N, jnp.int32)
    srcp = jnp.concatenate([src, pad]).reshape(ERP, 128)
    dstp = jnp.concatenate([dst, pad]).reshape(ERP, 128)

    ones = jnp.ones((128, 128), jnp.float32)
    z128 = jnp.zeros((NP, 128), jnp.float32)

    degp = _make_deg()(dstp, ones, z128)
    g1 = _mm1(degp, x, W1)
    agg1 = _make_agg()(g1, srcp, dstp, z128)
    y = _mid(degp, agg1, g1, b1)
    agg2 = _make_agg()(y, srcp, dstp, z128)
    return _final(degp, agg2, y, W2, b2)
